# Initial kernel scaffold; baseline (speedup 1.0000x reference)
import functools

import jax
import jax.numpy as jnp
from jax import lax
from jax.experimental import pallas as pl
from jax.experimental.pallas import tpu as pltpu
from jax.experimental.pallas import tpu_sc as plsc

VOCAB = 1000000
EMBED = 32
NUM_CLASSES = 100
BATCH = 4096
HIST = 200

NUM_CORES = 2
NUM_SUBCORES = 16
NUM_WORKERS = NUM_CORES * NUM_SUBCORES
B_PER_W = BATCH // NUM_WORKERS
C0 = 104
C1 = HIST - C0

_SCALE = 1.0 / HIST


def _pool_body(ids_hbm, table_hbm, out_hbm, idx_v, buf0, buf1, pooled_v, sem):
    wid = lax.axis_index("s") * NUM_CORES + lax.axis_index("c")
    base = wid * B_PER_W
    pltpu.sync_copy(ids_hbm.at[pl.ds(base * HIST, B_PER_W * HIST)], idx_v)

    def accumulate(buf, n, acc0, acc1):
        for i in range(n):
            acc0 = acc0 + buf[i, 0:16]
            acc1 = acc1 + buf[i, 16:32]
        return acc0, acc1

    def row_body(r, carry):
        off = pl.multiple_of(r * HIST, 8)
        cp0 = pltpu.async_copy(table_hbm.at[idx_v.at[pl.ds(off, C0)]], buf0, sem)
        cp1 = pltpu.async_copy(
            table_hbm.at[idx_v.at[pl.ds(off + C0, C1)]], buf1, sem
        )
        cp0.wait()
        cp1.wait()
        acc0 = jnp.zeros((16,), jnp.float32)
        acc1 = jnp.zeros((16,), jnp.float32)
        acc0, acc1 = accumulate(buf0, C0, acc0, acc1)
        acc0, acc1 = accumulate(buf1, C1, acc0, acc1)
        pooled_v[r, 0:16] = acc0 * _SCALE
        pooled_v[r, 16:32] = acc1 * _SCALE
        return carry

    lax.fori_loop(0, B_PER_W, row_body, 0)
    pltpu.sync_copy(pooled_v, out_hbm.at[pl.ds(base, B_PER_W)])


def _make_pool_kernel():
    mesh = plsc.VectorSubcoreMesh(
        core_axis_name="c",
        subcore_axis_name="s",
        num_cores=NUM_CORES,
        num_subcores=NUM_SUBCORES,
    )
    return pl.kernel(
        _pool_body,
        out_type=jax.ShapeDtypeStruct((BATCH, EMBED), jnp.float32),
        mesh=mesh,
        scratch_types=[
            pltpu.VMEM((B_PER_W * HIST,), jnp.int32),
            pltpu.VMEM((C0, EMBED), jnp.float32),
            pltpu.VMEM((C1, EMBED), jnp.float32),
            pltpu.VMEM((B_PER_W, EMBED), jnp.float32),
            pltpu.SemaphoreType.DMA,
        ],
    )


def _linear_body(pooled_ref, w_ref, b_ref, out_ref):
    out_ref[...] = (
        jnp.dot(pooled_ref[...], w_ref[...], preferred_element_type=jnp.float32)
        + b_ref[...]
    )


def kernel(input_ids, emb_table, fc_w, fc_b):
    ids_flat = input_ids.reshape(-1).astype(jnp.int32)
    pooled = _make_pool_kernel()(ids_flat, emb_table)
    out = pl.pallas_call(
        _linear_body,
        out_shape=jax.ShapeDtypeStruct((BATCH, NUM_CLASSES), jnp.float32),
    )(pooled, fc_w.T, fc_b[None, :])
    return out


# speedup vs baseline: 2.0548x; 2.0548x over previous
import functools

import jax
import jax.numpy as jnp
from jax import lax
from jax.experimental import pallas as pl
from jax.experimental.pallas import tpu as pltpu
from jax.experimental.pallas import tpu_sc as plsc

VOCAB = 1000000
EMBED = 32
NUM_CLASSES = 100
BATCH = 4096
HIST = 200

NUM_CORES = 2
NUM_SUBCORES = 16
NUM_WORKERS = NUM_CORES * NUM_SUBCORES
B_PER_W = BATCH // NUM_WORKERS
C0 = 104
C1 = HIST - C0

_SCALE = 1.0 / HIST


def _pool_body(ids_hbm, table_hbm, out_hbm, idx_v, buf0, buf1, pooled_v, sem):
    wid = lax.axis_index("s") * NUM_CORES + lax.axis_index("c")
    base = wid * B_PER_W
    pltpu.sync_copy(ids_hbm.at[pl.ds(base * HIST, B_PER_W * HIST)], idx_v)

    def accumulate(buf, n, acc0, acc1):
        for i in range(n):
            acc0 = acc0 + buf[i, 0:16]
            acc1 = acc1 + buf[i, 16:32]
        return acc0, acc1

    def row_body(r, carry):
        off = pl.multiple_of(r * HIST, 8)
        cp0 = pltpu.async_copy(table_hbm.at[idx_v.at[pl.ds(off, C0)]], buf0, sem)
        cp1 = pltpu.async_copy(
            table_hbm.at[idx_v.at[pl.ds(off + C0, C1)]], buf1, sem
        )
        cp0.wait()
        cp1.wait()
        acc0 = jnp.zeros((16,), jnp.float32)
        acc1 = jnp.zeros((16,), jnp.float32)
        acc0, acc1 = accumulate(buf0, C0, acc0, acc1)
        acc0, acc1 = accumulate(buf1, C1, acc0, acc1)
        pooled_v[r, 0:16] = acc0 * _SCALE
        pooled_v[r, 16:32] = acc1 * _SCALE
        return carry

    lax.fori_loop(0, B_PER_W, row_body, 0)
    pltpu.sync_copy(pooled_v, out_hbm.at[pl.ds(base, B_PER_W)])


def _make_pool_kernel():
    mesh = plsc.VectorSubcoreMesh(
        core_axis_name="c",
        subcore_axis_name="s",
        num_cores=NUM_CORES,
        num_subcores=NUM_SUBCORES,
    )
    return pl.kernel(
        _pool_body,
        out_type=jax.ShapeDtypeStruct((BATCH, EMBED), jnp.float32),
        mesh=mesh,
        scratch_types=[
            pltpu.VMEM((B_PER_W * HIST,), jnp.int32),
            pltpu.VMEM((C0, EMBED), jnp.float32),
            pltpu.VMEM((C1, EMBED), jnp.float32),
            pltpu.VMEM((B_PER_W, EMBED), jnp.float32),
            pltpu.SemaphoreType.DMA,
        ],
        compiler_params=pltpu.CompilerParams(use_tc_tiling_on_sc=False),
    )


def _linear_body(pooled_ref, w_ref, b_ref, out_ref):
    out_ref[...] = (
        jnp.dot(pooled_ref[...], w_ref[...], preferred_element_type=jnp.float32)
        + b_ref[...]
    )


def kernel(input_ids, emb_table, fc_w, fc_b):
    ids_flat = input_ids.reshape(-1).astype(jnp.int32)
    pooled = _make_pool_kernel()(ids_flat, emb_table)
    out = pl.pallas_call(
        _linear_body,
        out_shape=jax.ShapeDtypeStruct((BATCH, NUM_CLASSES), jnp.float32),
    )(pooled, fc_w.T, fc_b[None, :])
    return out
